# baseline (speedup 1.0000x reference)
import jax
import jax.numpy as jnp
from jax import lax
from jax.experimental import pallas as pl
from jax.experimental.pallas import tpu as pltpu
from jax.experimental.pallas import tpu_sc as plsc
import functools


@functools.partial(jax.jit, static_argnames=("seq_len",))
def _sc_copy(table, seq_len):
    V, D = table.shape
    NC, NS = 2, 16
    NW = NC * NS
    rows_per_w = seq_len // NW
    chunk = 16
    nbuf = 6
    n_chunks = rows_per_w // chunk

    mesh = plsc.VectorSubcoreMesh(
        core_axis_name="c", subcore_axis_name="s", num_cores=NC, num_subcores=NS
    )

    @functools.partial(
        pl.kernel,
        out_type=jax.ShapeDtypeStruct((seq_len, D), table.dtype),
        mesh=mesh,
        scratch_types=(
            [pltpu.VMEM((chunk, D), table.dtype) for _ in range(nbuf)]
            + [pltpu.SemaphoreType.DMA for _ in range(2 * nbuf)]
        ),
    )
    def k(table_hbm, out_hbm, *scratch):
        bufs = scratch[:nbuf]
        gsems = scratch[nbuf : 2 * nbuf]
        ssems = scratch[2 * nbuf :]
        wid = lax.axis_index("s") * NC + lax.axis_index("c")
        base = wid * rows_per_w

        gathers = [None] * n_chunks
        scatters = [None] * n_chunks
        for i in range(min(nbuf, n_chunks)):
            gathers[i] = pltpu.async_copy(
                table_hbm.at[pl.ds(base + i * chunk, chunk)], bufs[i], gsems[i]
            )
        for i in range(n_chunks):
            b = i % nbuf
            gathers[i].wait()
            scatters[i] = pltpu.async_copy(
                bufs[b], out_hbm.at[pl.ds(base + i * chunk, chunk)], ssems[b]
            )
            nxt = i + nbuf
            if nxt < n_chunks:
                scatters[i].wait()
                gathers[nxt] = pltpu.async_copy(
                    table_hbm.at[pl.ds(base + nxt * chunk, chunk)],
                    bufs[b],
                    gsems[b],
                )
        for i in range(max(0, n_chunks - nbuf), n_chunks):
            if scatters[i] is not None:
                scatters[i].wait()

    return k(table)


def kernel(x, table):
    seq_len = x.shape[1]
    emb = _sc_copy(table, seq_len)
    return emb[None, :, :]


# speedup vs baseline: 1.0057x; 1.0057x over previous
import jax
import jax.numpy as jnp
from jax import lax
from jax.experimental import pallas as pl
from jax.experimental.pallas import tpu as pltpu
from jax.experimental.pallas import tpu_sc as plsc
import functools


@functools.partial(jax.jit, static_argnames=("seq_len",))
def _sc_copy(table, seq_len):
    V, D = table.shape
    NC, NS = 2, 16
    NW = NC * NS
    rows_per_w = seq_len // NW
    chunk = 32
    nbuf = 3
    n_chunks = rows_per_w // chunk

    mesh = plsc.VectorSubcoreMesh(
        core_axis_name="c", subcore_axis_name="s", num_cores=NC, num_subcores=NS
    )

    @functools.partial(
        pl.kernel,
        out_type=jax.ShapeDtypeStruct((seq_len, D), table.dtype),
        mesh=mesh,
        scratch_types=(
            [pltpu.VMEM((chunk, D), table.dtype) for _ in range(nbuf)]
            + [pltpu.SemaphoreType.DMA for _ in range(2 * nbuf)]
        ),
    )
    def k(table_hbm, out_hbm, *scratch):
        bufs = scratch[:nbuf]
        gsems = scratch[nbuf : 2 * nbuf]
        ssems = scratch[2 * nbuf :]
        wid = lax.axis_index("s") * NC + lax.axis_index("c")
        base = wid * rows_per_w

        gathers = [None] * n_chunks
        scatters = [None] * n_chunks
        for i in range(min(nbuf, n_chunks)):
            gathers[i] = pltpu.async_copy(
                table_hbm.at[pl.ds(base + i * chunk, chunk)], bufs[i], gsems[i]
            )
        for i in range(n_chunks):
            b = i % nbuf
            gathers[i].wait()
            scatters[i] = pltpu.async_copy(
                bufs[b], out_hbm.at[pl.ds(base + i * chunk, chunk)], ssems[b]
            )
            nxt = i + nbuf
            if nxt < n_chunks:
                scatters[i].wait()
                gathers[nxt] = pltpu.async_copy(
                    table_hbm.at[pl.ds(base + nxt * chunk, chunk)],
                    bufs[b],
                    gsems[b],
                )
        for i in range(max(0, n_chunks - nbuf), n_chunks):
            if scatters[i] is not None:
                scatters[i].wait()

    return k(table)


def kernel(x, table):
    seq_len = x.shape[1]
    emb = _sc_copy(table, seq_len)
    return emb[None, :, :]
